# baseline (speedup 1.0000x reference)
import jax
import jax.numpy as jnp
from jax.experimental import pallas as pl
from jax.experimental.pallas import tpu as pltpu

_EPS = 0.01
_N = 4096
_BLK = 256


def _toz_body(x_ref, o_ref):
    i = pl.program_id(0)
    o_ref[...] = jnp.zeros((_BLK, 1, 64, 64), jnp.float32)

    @pl.when(i < 16)
    def _():
        slab = jnp.where(
            jax.lax.broadcasted_iota(jnp.int32, (64, 1, 1, 64), 0)
            == jax.lax.broadcasted_iota(jnp.int32, (64, 1, 1, 64), 3),
            _EPS, 0.0).astype(jnp.float32)
        for g in range(3):
            o_ref[pl.ds(64 * g + 1, 64), :, pl.ds(4 * i + g, 1), :] = slab
        o_ref[pl.ds(193, 63), :, pl.ds(4 * i + 3, 1), :] = slab[:63]

    @pl.when(i > 0)
    def _():
        o_ref[pl.ds(0, 1), :, pl.ds(4 * i - 1, 1), :] = jnp.where(
            jax.lax.broadcasted_iota(jnp.int32, (1, 1, 1, 64), 3) == 63,
            _EPS, 0.0).astype(jnp.float32)

    @pl.when(i == 0)
    def _():
        o_ref[pl.ds(0, 1), :, :, :] = x_ref[...]


def kernel(x):
    grid = (_N + 1 + _BLK - 1) // _BLK
    out = pl.pallas_call(
        _toz_body,
        grid=(grid,),
        in_specs=[pl.BlockSpec((1, 1, 64, 64), lambda i: (0, 0, 0, 0))],
        out_specs=pl.BlockSpec((_BLK, 1, 64, 64), lambda i: (i, 0, 0, 0)),
        out_shape=jax.ShapeDtypeStruct((_N + 1, 1, 64, 64), jnp.float32),
    )(x)
    return out
